# baseline (speedup 1.0000x reference)
import functools

import jax
import jax.numpy as jnp
from jax import lax
from jax.experimental import pallas as pl
from jax.experimental.pallas import tpu as pltpu
from jax.experimental.pallas import tpu_sc as plsc

_VOCAB = 1000
_BINS = 1024
_NPRIV = 16
_BSZ = 16
_SEQ = 2048
_L = 16
_PAT = 1024


def _sc_majority(ids_hbm, rows_hbm, tok_ref, counts_ref, pat_ref):
    wid = lax.axis_index("s") * 2 + lax.axis_index("c")

    @pl.when(wid < _BSZ)
    def _():
        lane = lax.iota(jnp.int32, _L)
        zeros = jnp.zeros((_L,), jnp.float32)
        ones = jnp.ones((_L,), jnp.float32)

        pltpu.sync_copy(ids_hbm.at[wid], tok_ref)

        def zero_body(k, c):
            counts_ref[pl.ds(k * _L, _L)] = zeros
            return c

        lax.fori_loop(0, (_NPRIV * _BINS) // _L, zero_body, 0, unroll=16)
        counts_ref[pl.ds(0, _L)] = jnp.where(lane == 1, 0.5, 0.0).astype(
            jnp.float32)

        def scat_body(i, c):
            tok = tok_ref[pl.ds(i * _L, _L)]
            valid = (tok != 0) & (tok != 1)
            idx = tok + lane * _BINS
            plsc.addupdate_scatter(counts_ref, [idx], ones, mask=valid)
            return c

        lax.fori_loop(0, _SEQ // _L, scat_body, 0, unroll=8)

        def red_body(j, carry):
            bv, bi = carry
            v = counts_ref[pl.ds(j * _L, _L)]
            for a in range(1, _NPRIV):
                v = v + counts_ref[pl.ds(a * _BINS + j * _L, _L)]
            idv = j * _L + lane
            upd = v > bv
            return jnp.where(upd, v, bv), jnp.where(upd, idv, bi)

        bv0 = jnp.full((_L,), -1.0, jnp.float32)
        bi0 = jnp.zeros((_L,), jnp.int32)
        bv, bi = lax.fori_loop(0, _BINS // _L, red_body, (bv0, bi0), unroll=2)

        m = jnp.max(bv)
        cand = jnp.where(bv == m, bi, jnp.int32(1 << 30))
        p = jnp.min(cand)

        neg = jnp.full((_L,), -6.0, jnp.float32)
        for s in range(_PAT // _L):
            pat_ref[pl.ds(s * _L, _L)] = neg
        plsc.store_scatter(
            pat_ref, [jnp.full((_L,), p, jnp.int32)],
            jnp.full((_L,), 6.0, jnp.float32), mask=lane == 0)

        pltpu.sync_copy(pat_ref, rows_hbm.at[pl.ds(wid * _PAT, _PAT)])


_sc_rows = functools.partial(
    pl.kernel,
    mesh=plsc.VectorSubcoreMesh(core_axis_name="c", subcore_axis_name="s"),
    out_type=jax.ShapeDtypeStruct((_BSZ * _PAT,), jnp.float32),
    compiler_params=pltpu.CompilerParams(needs_layout_passes=False),
    scratch_types=[
        pltpu.VMEM((_SEQ,), jnp.int32),
        pltpu.VMEM((_NPRIV * _BINS,), jnp.float32),
        pltpu.VMEM((_PAT,), jnp.float32),
    ],
)(_sc_majority)


@jax.jit
def kernel(input_ids):
    flat = _sc_rows(input_ids)
    rows = flat.reshape(_BSZ, _PAT)[:, :_VOCAB]
    return jnp.broadcast_to(rows[:, None, :], (_BSZ, _SEQ, _VOCAB))


# speedup vs baseline: 1.0037x; 1.0037x over previous
import functools

import jax
import jax.numpy as jnp
from jax import lax
from jax.experimental import pallas as pl
from jax.experimental.pallas import tpu as pltpu
from jax.experimental.pallas import tpu_sc as plsc

_VOCAB = 1000
_BINS = 1024
_NPRIV = 16
_BSZ = 16
_SEQ = 2048
_L = 16
_PAT = 1024


def _sc_majority(ids_hbm, rows_hbm, tok_ref, counts_ref, pat_ref):
    wid = lax.axis_index("s") * 2 + lax.axis_index("c")

    @pl.when(wid < _BSZ)
    def _():
        lane = lax.iota(jnp.int32, _L)
        zeros = jnp.zeros((_L,), jnp.float32)
        ones = jnp.ones((_L,), jnp.float32)

        pltpu.sync_copy(ids_hbm.at[wid], tok_ref)

        def zero_body(k, c):
            counts_ref[pl.ds(k * _L, _L)] = zeros
            return c

        lax.fori_loop(0, (_NPRIV * _BINS) // _L, zero_body, 0, unroll=8)
        counts_ref[pl.ds(0, _L)] = jnp.where(lane == 1, 0.5, 0.0).astype(
            jnp.float32)

        def scat_body(i, c):
            tok = tok_ref[pl.ds(i * _L, _L)]
            valid = (tok != 0) & (tok != 1)
            idx = tok + lane * _BINS
            plsc.addupdate_scatter(counts_ref, [idx], ones, mask=valid)
            return c

        lax.fori_loop(0, _SEQ // _L, scat_body, 0, unroll=4)

        def red_body(j, carry):
            bv, bi = carry
            v = counts_ref[pl.ds(j * _L, _L)]
            for a in range(1, _NPRIV):
                v = v + counts_ref[pl.ds(a * _BINS + j * _L, _L)]
            idv = j * _L + lane
            upd = v > bv
            return jnp.where(upd, v, bv), jnp.where(upd, idv, bi)

        bv0 = jnp.full((_L,), -1.0, jnp.float32)
        bi0 = jnp.zeros((_L,), jnp.int32)
        bv, bi = lax.fori_loop(0, _BINS // _L, red_body, (bv0, bi0))

        m = jnp.max(bv)
        cand = jnp.where(bv == m, bi, jnp.int32(1 << 30))
        p = jnp.min(cand)

        neg = jnp.full((_L,), -6.0, jnp.float32)
        for s in range(_PAT // _L):
            pat_ref[pl.ds(s * _L, _L)] = neg
        plsc.store_scatter(
            pat_ref, [jnp.full((_L,), p, jnp.int32)],
            jnp.full((_L,), 6.0, jnp.float32), mask=lane == 0)

        pltpu.sync_copy(pat_ref, rows_hbm.at[pl.ds(wid * _PAT, _PAT)])


_sc_rows = functools.partial(
    pl.kernel,
    mesh=plsc.VectorSubcoreMesh(core_axis_name="c", subcore_axis_name="s"),
    out_type=jax.ShapeDtypeStruct((_BSZ * _PAT,), jnp.float32),
    compiler_params=pltpu.CompilerParams(needs_layout_passes=False),
    scratch_types=[
        pltpu.VMEM((_SEQ,), jnp.int32),
        pltpu.VMEM((_NPRIV * _BINS,), jnp.float32),
        pltpu.VMEM((_PAT,), jnp.float32),
    ],
)(_sc_majority)


@jax.jit
def kernel(input_ids):
    flat = _sc_rows(input_ids)
    rows = flat.reshape(_BSZ, _PAT)[:, :_VOCAB]
    return jnp.broadcast_to(rows[:, None, :], (_BSZ, _SEQ, _VOCAB))


# speedup vs baseline: 1.0205x; 1.0168x over previous
import functools

import jax
import jax.numpy as jnp
from jax import lax
from jax.experimental import pallas as pl
from jax.experimental.pallas import tpu as pltpu
from jax.experimental.pallas import tpu_sc as plsc

_VOCAB = 1000
_BINS = 1024
_BSZ = 16
_SEQ = 2048
_L = 16
_PAT = 1024


def _sc_majority(ids_hbm, rows_hbm, tok_ref, counts_ref, pat_ref):
    wid = lax.axis_index("s") * 2 + lax.axis_index("c")

    @pl.when(wid < _BSZ)
    def _():
        lane = lax.iota(jnp.int32, _L)
        zeros = jnp.zeros((_L,), jnp.float32)
        ones = jnp.ones((_L,), jnp.float32)

        pltpu.sync_copy(ids_hbm.at[wid], tok_ref)

        def zero_body(k, c):
            counts_ref[pl.ds(k * _L, _L)] = zeros
            return c

        lax.fori_loop(0, _BINS // _L, zero_body, 0, unroll=8)
        counts_ref[pl.ds(0, _L)] = jnp.where(lane == 1, 0.5, 0.0).astype(
            jnp.float32)

        def scat_body(i, c):
            tok = tok_ref[pl.ds(i * _L, _L)]
            valid = (tok != 0) & (tok != 1)
            plsc.addupdate_scatter(counts_ref, [tok], ones, mask=valid)
            return c

        lax.fori_loop(0, _SEQ // _L, scat_body, 0, unroll=4)

        def red_body(j, carry):
            bv, bi = carry
            v = counts_ref[pl.ds(j * _L, _L)]
            idv = j * _L + lane
            upd = v > bv
            return jnp.where(upd, v, bv), jnp.where(upd, idv, bi)

        bv0 = jnp.full((_L,), -1.0, jnp.float32)
        bi0 = jnp.zeros((_L,), jnp.int32)
        bv, bi = lax.fori_loop(0, _BINS // _L, red_body, (bv0, bi0))

        m = jnp.max(bv)
        cand = jnp.where(bv == m, bi, jnp.int32(1 << 30))
        p = jnp.min(cand)

        neg = jnp.full((_L,), -6.0, jnp.float32)
        for s in range(_PAT // _L):
            pat_ref[pl.ds(s * _L, _L)] = neg
        plsc.store_scatter(
            pat_ref, [jnp.full((_L,), p, jnp.int32)],
            jnp.full((_L,), 6.0, jnp.float32), mask=lane == 0)

        pltpu.sync_copy(pat_ref, rows_hbm.at[pl.ds(wid * _PAT, _PAT)])


_sc_rows = functools.partial(
    pl.kernel,
    mesh=plsc.VectorSubcoreMesh(core_axis_name="c", subcore_axis_name="s"),
    out_type=jax.ShapeDtypeStruct((_BSZ * _PAT,), jnp.float32),
    compiler_params=pltpu.CompilerParams(needs_layout_passes=False),
    scratch_types=[
        pltpu.VMEM((_SEQ,), jnp.int32),
        pltpu.VMEM((_BINS,), jnp.float32),
        pltpu.VMEM((_PAT,), jnp.float32),
    ],
)(_sc_majority)


@jax.jit
def kernel(input_ids):
    flat = _sc_rows(input_ids)
    rows = flat.reshape(_BSZ, _PAT)[:, :_VOCAB]
    return jnp.broadcast_to(rows[:, None, :], (_BSZ, _SEQ, _VOCAB))


# speedup vs baseline: 1.0247x; 1.0041x over previous
import functools

import jax
import jax.numpy as jnp
from jax import lax
from jax.experimental import pallas as pl
from jax.experimental.pallas import tpu as pltpu
from jax.experimental.pallas import tpu_sc as plsc

_VOCAB = 1000
_BINS = 1024
_BSZ = 16
_SEQ = 2048
_L = 16
_PAT = 1024


def _sc_majority(ids_hbm, rows_hbm, tok_ref, counts_ref, pat_ref):
    wid = lax.axis_index("s") * 2 + lax.axis_index("c")

    @pl.when(wid < _BSZ)
    def _():
        lane = lax.iota(jnp.int32, _L)
        zeros = jnp.zeros((_L,), jnp.float32)
        ones = jnp.ones((_L,), jnp.float32)

        pltpu.sync_copy(ids_hbm.at[wid], tok_ref)

        def zero_body(k, c):
            counts_ref[pl.ds(k * _L, _L)] = zeros
            return c

        lax.fori_loop(0, _BINS // _L, zero_body, 0, unroll=16)
        counts_ref[pl.ds(0, _L)] = jnp.where(lane == 1, 0.5, 0.0).astype(
            jnp.float32)

        def scat_body(i, c):
            tok = tok_ref[pl.ds(i * _L, _L)]
            valid = (tok != 0) & (tok != 1)
            plsc.addupdate_scatter(counts_ref, [tok], ones, mask=valid)
            return c

        lax.fori_loop(0, _SEQ // _L, scat_body, 0, unroll=8)

        def red_body(j, carry):
            bv, bi = carry
            v = counts_ref[pl.ds(j * _L, _L)]
            idv = j * _L + lane
            upd = v > bv
            return jnp.where(upd, v, bv), jnp.where(upd, idv, bi)

        bv0 = jnp.full((_L,), -1.0, jnp.float32)
        bi0 = jnp.zeros((_L,), jnp.int32)
        bv, bi = lax.fori_loop(0, _BINS // _L, red_body, (bv0, bi0), unroll=4)

        m = jnp.max(bv)
        cand = jnp.where(bv == m, bi, jnp.int32(1 << 30))
        p = jnp.min(cand)

        neg = jnp.full((_L,), -6.0, jnp.float32)
        for s in range(_PAT // _L):
            pat_ref[pl.ds(s * _L, _L)] = neg
        plsc.store_scatter(
            pat_ref, [jnp.full((_L,), p, jnp.int32)],
            jnp.full((_L,), 6.0, jnp.float32), mask=lane == 0)

        pltpu.sync_copy(pat_ref, rows_hbm.at[pl.ds(wid * _PAT, _PAT)])


_sc_rows = functools.partial(
    pl.kernel,
    mesh=plsc.VectorSubcoreMesh(core_axis_name="c", subcore_axis_name="s"),
    out_type=jax.ShapeDtypeStruct((_BSZ * _PAT,), jnp.float32),
    compiler_params=pltpu.CompilerParams(needs_layout_passes=False),
    scratch_types=[
        pltpu.VMEM((_SEQ,), jnp.int32),
        pltpu.VMEM((_BINS,), jnp.float32),
        pltpu.VMEM((_PAT,), jnp.float32),
    ],
)(_sc_majority)


@jax.jit
def kernel(input_ids):
    flat = _sc_rows(input_ids)
    rows = flat.reshape(_BSZ, _PAT)[:, :_VOCAB]
    return jnp.broadcast_to(rows[:, None, :], (_BSZ, _SEQ, _VOCAB))
